# Initial kernel scaffold; baseline (speedup 1.0000x reference)
import jax
import jax.numpy as jnp
from jax.experimental import pallas as pl


def kernel(word, fasttext, glove, W_ft, b_ft, W_gl, b_gl, ft_scalar, gl_scalar):
    raise NotImplementedError("write your pallas kernel here")


# speedup vs baseline: 6.6267x; 6.6267x over previous
import functools

import jax
import jax.numpy as jnp
from jax import lax
from jax.experimental import pallas as pl
from jax.experimental.pallas import tpu as pltpu
from jax.experimental.pallas import tpu_sc as plsc

DIM_FT = 300
DIM_GL = 50
DIM_OUT = 256

VB = 2000


def _table_body(ft_ref, gl_ref, wft_ref, wgl_ref, bias_ref, out_ref):
    acc = jnp.dot(ft_ref[...], wft_ref[...], preferred_element_type=jnp.float32)
    acc += jnp.dot(gl_ref[...], wgl_ref[...], preferred_element_type=jnp.float32)
    acc += bias_ref[...]
    out_ref[...] = 1.0 / (1.0 + jnp.exp(-acc))


def _build_table(fasttext, glove, w_ft_s, w_gl_s, bias):
    vocab = fasttext.shape[0]
    return pl.pallas_call(
        _table_body,
        grid=(vocab // VB,),
        in_specs=[
            pl.BlockSpec((VB, DIM_FT), lambda i: (i, 0)),
            pl.BlockSpec((VB, DIM_GL), lambda i: (i, 0)),
            pl.BlockSpec((DIM_FT, DIM_OUT), lambda i: (0, 0)),
            pl.BlockSpec((DIM_GL, DIM_OUT), lambda i: (0, 0)),
            pl.BlockSpec((1, DIM_OUT), lambda i: (0, 0)),
        ],
        out_specs=pl.BlockSpec((VB, DIM_OUT), lambda i: (i, 0)),
        out_shape=jax.ShapeDtypeStruct((vocab, DIM_OUT), jnp.float32),
    )(fasttext, glove, w_ft_s, w_gl_s, bias)


def _gather_rows(table, idx3):
    nw, nchunk, cs = idx3.shape
    info = plsc.get_sparse_core_info()
    nc = info.num_cores
    mesh = plsc.VectorSubcoreMesh(core_axis_name="c", subcore_axis_name="s")

    @functools.partial(
        pl.kernel,
        mesh=mesh,
        out_type=jax.ShapeDtypeStruct((nw * nchunk * cs, DIM_OUT), jnp.float32),
        scratch_types=[
            pltpu.VMEM((nchunk, cs), jnp.int32),
            pltpu.VMEM((cs, DIM_OUT), jnp.float32),
            pltpu.SemaphoreType.DMA,
        ],
    )
    def k(table_hbm, idx_hbm, out_hbm, idx_v, buf, sem):
        wid = lax.axis_index("s") * nc + lax.axis_index("c")
        base = wid * (nchunk * cs)
        pltpu.sync_copy(idx_hbm.at[wid], idx_v)

        def body(ci, carry):
            pltpu.async_copy(table_hbm.at[idx_v.at[ci]], buf, sem).wait()
            pltpu.sync_copy(buf, out_hbm.at[pl.ds(base + ci * cs, cs)])
            return carry

        lax.fori_loop(0, nchunk, body, 0)

    return k(table, idx3)


def kernel(word, fasttext, glove, W_ft, b_ft, W_gl, b_gl, ft_scalar, gl_scalar):
    fts = ft_scalar[0]
    gls = gl_scalar[0]
    bias = (fts * b_ft + gls * b_gl).reshape(1, DIM_OUT)
    table = _build_table(fasttext, glove, W_ft * fts, W_gl * gls, bias)

    b, l = word.shape
    ntok = b * l
    nw = 32
    cs = 128
    nchunk = ntok // (nw * cs)
    idx3 = word.astype(jnp.int32).reshape(nw, nchunk, cs)
    out_flat = _gather_rows(table, idx3)
    return out_flat.reshape(b, l, DIM_OUT)


# speedup vs baseline: 6.8749x; 1.0374x over previous
import functools

import jax
import jax.numpy as jnp
from jax import lax
from jax.experimental import pallas as pl
from jax.experimental.pallas import tpu as pltpu
from jax.experimental.pallas import tpu_sc as plsc

DIM_FT = 300
DIM_GL = 50
DIM_OUT = 256

VB = 2000


def _table_body(ft_ref, gl_ref, wft_ref, wgl_ref, bias_ref, out_ref):
    acc = jnp.dot(ft_ref[...], wft_ref[...], preferred_element_type=jnp.float32)
    acc += jnp.dot(gl_ref[...], wgl_ref[...], preferred_element_type=jnp.float32)
    acc += bias_ref[...]
    out_ref[...] = 1.0 / (1.0 + jnp.exp(-acc))


def _build_table(fasttext, glove, w_ft_s, w_gl_s, bias):
    vocab = fasttext.shape[0]
    return pl.pallas_call(
        _table_body,
        grid=(vocab // VB,),
        in_specs=[
            pl.BlockSpec((VB, DIM_FT), lambda i: (i, 0)),
            pl.BlockSpec((VB, DIM_GL), lambda i: (i, 0)),
            pl.BlockSpec((DIM_FT, DIM_OUT), lambda i: (0, 0)),
            pl.BlockSpec((DIM_GL, DIM_OUT), lambda i: (0, 0)),
            pl.BlockSpec((1, DIM_OUT), lambda i: (0, 0)),
        ],
        out_specs=pl.BlockSpec((VB, DIM_OUT), lambda i: (i, 0)),
        out_shape=jax.ShapeDtypeStruct((vocab, DIM_OUT), jnp.float32),
    )(fasttext, glove, w_ft_s, w_gl_s, bias)


def _gather_rows(table, idx3):
    nw, nchunk, cs = idx3.shape
    info = plsc.get_sparse_core_info()
    nc = info.num_cores
    mesh = plsc.VectorSubcoreMesh(core_axis_name="c", subcore_axis_name="s")

    @functools.partial(
        pl.kernel,
        mesh=mesh,
        out_type=jax.ShapeDtypeStruct((nw * nchunk * cs, DIM_OUT), jnp.float32),
        scratch_types=[
            pltpu.VMEM((nchunk, cs), jnp.int32),
            pltpu.VMEM((cs, DIM_OUT), jnp.float32),
            pltpu.VMEM((cs, DIM_OUT), jnp.float32),
            pltpu.SemaphoreType.DMA,
            pltpu.SemaphoreType.DMA,
        ],
    )
    def k(table_hbm, idx_hbm, out_hbm, idx_v, buf0, buf1, sem0, sem1):
        wid = lax.axis_index("s") * nc + lax.axis_index("c")
        base = wid * (nchunk * cs)
        pltpu.sync_copy(idx_hbm.at[wid], idx_v)
        pltpu.async_copy(table_hbm.at[idx_v.at[0]], buf0, sem0)
        pltpu.async_copy(table_hbm.at[idx_v.at[1]], buf1, sem1)

        def body(i, carry):
            g = i * 2
            pltpu.make_async_copy(table_hbm.at[idx_v.at[0]], buf0, sem0).wait()
            pltpu.sync_copy(buf0, out_hbm.at[pl.ds(base + g * cs, cs)])
            nxt0 = jnp.minimum(g + 2, nchunk - 1)
            pltpu.async_copy(table_hbm.at[idx_v.at[nxt0]], buf0, sem0)
            pltpu.make_async_copy(table_hbm.at[idx_v.at[1]], buf1, sem1).wait()
            pltpu.sync_copy(buf1, out_hbm.at[pl.ds(base + (g + 1) * cs, cs)])
            nxt1 = jnp.minimum(g + 3, nchunk - 1)
            pltpu.async_copy(table_hbm.at[idx_v.at[nxt1]], buf1, sem1)
            return carry

        lax.fori_loop(0, nchunk // 2, body, 0)
        pltpu.make_async_copy(table_hbm.at[idx_v.at[0]], buf0, sem0).wait()
        pltpu.make_async_copy(table_hbm.at[idx_v.at[1]], buf1, sem1).wait()

    return k(table, idx3)


def kernel(word, fasttext, glove, W_ft, b_ft, W_gl, b_gl, ft_scalar, gl_scalar):
    fts = ft_scalar[0]
    gls = gl_scalar[0]
    bias = (fts * b_ft + gls * b_gl).reshape(1, DIM_OUT)
    table = _build_table(fasttext, glove, W_ft * fts, W_gl * gls, bias)

    b, l = word.shape
    ntok = b * l
    nw = 32
    cs = 128
    nchunk = ntok // (nw * cs)
    idx3 = word.astype(jnp.int32).reshape(nw, nchunk, cs)
    out_flat = _gather_rows(table, idx3)
    return out_flat.reshape(b, l, DIM_OUT)
